# baseline (speedup 1.0000x reference)
import functools

import jax
import jax.numpy as jnp
from jax import lax
from jax.experimental import pallas as pl
from jax.experimental.pallas import tpu as pltpu
from jax.experimental.pallas import tpu_sc as plsc

B, T, D = 4, 2048, 1024
NW = 32
TW = T // NW
R = 8
K = TW // R
CD = D // 16
NV = R * CD
P = 3


def _sc_body(x_hbm, tab_hbm, out_hbm,
             tb0, tb1, xb0, xb1, xb2,
             st0, st1, sx0, sx1, sx2, so0, so1, so2):
    tbuf = (tb0, tb1)
    xbuf = (xb0, xb1, xb2)
    sem_t = (st0, st1)
    sem_x = (sx0, sx1, sx2)
    sem_o = (so0, so1, so2)

    wid = lax.axis_index("s") * 2 + lax.axis_index("c")
    t_base = wid * TW

    hx = [None] * K
    ho = [None] * K
    ht = [None] * K

    def start_x(k):
        p = k % P
        t0 = t_base + k * R
        hx[k] = pltpu.async_copy(x_hbm.at[:, pl.ds(t0, R)], xbuf[p],
                                 sem_x[p])

    def start_t(k):
        t0 = t_base + k * R
        ht[k] = pltpu.async_copy(tab_hbm.at[pl.ds(t0, R)], tbuf[k % 2],
                                 sem_t[k % 2])

    start_t(0)
    start_t(1)
    start_x(0)
    start_x(1)

    for k in range(K):
        p = k % P
        hx[k].wait()
        ht[k].wait()
        tb = tbuf[k % 2]
        xs = xbuf[p]

        t0 = t_base + k * R
        hob = []
        for b in range(B):
            @plsc.parallel_loop(0, NV, unroll=8)
            def add(v, b=b):
                r = v // CD
                off = pl.ds((v % CD) * 16, 16)
                xs[b, r, off] = xs[b, r, off] + tb[r, off]

            hob.append(pltpu.async_copy(
                xs.at[b], out_hbm.at[b, pl.ds(t0, R)], sem_o[p]))
        ho[k] = hob
        if k + 2 < K:
            if k >= 1:
                for h in ho[k - 1]:
                    h.wait()
            start_x(k + 2)
            start_t(k + 2)

    for k in (K - 3, K - 2, K - 1):
        for h in ho[k]:
            h.wait()


@jax.jit
def _pos_add(x, pos_table):
    mesh = plsc.VectorSubcoreMesh(core_axis_name="c", subcore_axis_name="s")
    f = functools.partial(
        pl.kernel,
        mesh=mesh,
        out_type=jax.ShapeDtypeStruct((B, T, D), jnp.float32),
        scratch_types=(
            [pltpu.VMEM((R, D), jnp.float32)] * 2
            + [pltpu.VMEM((B, R, D), jnp.float32)] * 3
            + [pltpu.SemaphoreType.DMA] * 8
        ),
    )(_sc_body)
    return f(x, pos_table)


def kernel(x, pos_table):
    return _pos_add(x, pos_table)


# speedup vs baseline: 1.0431x; 1.0431x over previous
import functools

import jax
import jax.numpy as jnp
from jax import lax
from jax.experimental import pallas as pl
from jax.experimental.pallas import tpu as pltpu
from jax.experimental.pallas import tpu_sc as plsc

B, T, D = 4, 2048, 1024
NW = 32
TW = T // NW
R = 8
K = TW // R
CD = D // 16
NV = R * CD
P = 3


def _sc_body(x_hbm, tab_hbm, out_hbm,
             tb0, tb1, xb0, xb1, xb2,
             st0, st1, sx0, sx1, sx2, so0, so1, so2):
    tbuf = (tb0, tb1)
    xbuf = (xb0, xb1, xb2)
    sem_t = (st0, st1)
    sem_x = (sx0, sx1, sx2)
    sem_o = (so0, so1, so2)

    wid = lax.axis_index("s") * 2 + lax.axis_index("c")
    t_base = wid * TW

    hx = [None] * K
    ho = [None] * K
    ht = [None] * K

    def start_x(k):
        p = k % P
        t0 = t_base + k * R
        hx[k] = pltpu.async_copy(x_hbm.at[:, pl.ds(t0, R)], xbuf[p],
                                 sem_x[p])

    def start_t(k):
        t0 = t_base + k * R
        ht[k] = pltpu.async_copy(tab_hbm.at[pl.ds(t0, R)], tbuf[k % 2],
                                 sem_t[k % 2])

    start_t(0)
    start_t(1)
    start_x(0)
    start_x(1)

    for k in range(K):
        p = k % P
        hx[k].wait()
        ht[k].wait()
        tb = tbuf[k % 2]
        xs = xbuf[p]

        @plsc.parallel_loop(0, NV, unroll=8)
        def add(v):
            r = v // CD
            off = pl.ds((v % CD) * 16, 16)
            tv = tb[r, off]
            for b in range(B):
                xs[b, r, off] = xs[b, r, off] + tv

        t0 = t_base + k * R
        ho[k] = pltpu.async_copy(xs, out_hbm.at[:, pl.ds(t0, R)],
                                 sem_o[p])
        if k + 2 < K:
            if k >= 1:
                ho[k - 1].wait()
            start_x(k + 2)
            start_t(k + 2)

    for k in (K - 3, K - 2, K - 1):
        ho[k].wait()


@jax.jit
def _pos_add(x, pos_table):
    mesh = plsc.VectorSubcoreMesh(core_axis_name="c", subcore_axis_name="s")
    f = functools.partial(
        pl.kernel,
        mesh=mesh,
        out_type=jax.ShapeDtypeStruct((B, T, D), jnp.float32),
        scratch_types=(
            [pltpu.VMEM((R, D), jnp.float32)] * 2
            + [pltpu.VMEM((B, R, D), jnp.float32)] * 3
            + [pltpu.SemaphoreType.DMA] * 8
        ),
    )(_sc_body)
    return f(x, pos_table)


def kernel(x, pos_table):
    return _pos_add(x, pos_table)
